# Initial kernel scaffold; baseline (speedup 1.0000x reference)
import jax
import jax.numpy as jnp
from jax.experimental import pallas as pl


def kernel(features1, features2, features3, features4, features5, edge_index, Wf1, bf1, Wf2, bf2, Wf3, bf3, Wf4, bf4, Wf5, bf5, w1, w2, w3, w4, w5, Wg1, bg1, Wg2, bg2, Wb1, bb1, Wb2, bb2):
    raise NotImplementedError("write your pallas kernel here")


# speedup vs baseline: 3.5703x; 3.5703x over previous
import functools

import jax
import jax.numpy as jnp
from jax import lax
from jax.experimental import pallas as pl
from jax.experimental.pallas import tpu as pltpu
from jax.experimental.pallas import tpu_sc as plsc

N = 10000
E = 320000
NPAD = 10240
EROWS = 2560
EPAD = EROWS * 128 - E
NC, NS = 2, 16
RPW = EROWS // (NC * NS)
RPT = EROWS // NS
ZR = NPAD // NS

_MESH = dict(core_axis_name="c", subcore_axis_name="s", num_cores=NC,
             num_subcores=NS)


def _sc_degrees_spmem(srcdst_rows):
    @functools.partial(
        pl.kernel,
        out_type=jax.ShapeDtypeStruct((2, NPAD), jnp.float32),
        mesh=plsc.VectorSubcoreMesh(**_MESH),
        scratch_types=[
            pltpu.VMEM((RPT, 128), jnp.int32),
            pltpu.VMEM((128,), jnp.float32),
            pltpu.VMEM((ZR,), jnp.float32),
            pltpu.VMEM_SHARED((NPAD,), jnp.float32),
        ],
    )
    def deg_kernel(se_hbm, out_hbm, idx_v, ones_v, z_v, acc_sh):
        c = lax.axis_index("c")
        t = lax.axis_index("s")

        @pl.loop(0, 8)
        def _(q):
            ones_v[pl.ds(q * 16, 16)] = jnp.full((16,), 1.0, jnp.float32)

        @pl.loop(0, ZR // 16)
        def _(q):
            z_v[pl.ds(q * 16, 16)] = jnp.zeros((16,), jnp.float32)

        pltpu.sync_copy(z_v, acc_sh.at[pl.ds(t * ZR, ZR)])
        plsc.subcore_barrier()

        pltpu.sync_copy(se_hbm.at[pl.ds(c * EROWS + t * RPT, RPT)], idx_v)

        @pl.loop(0, RPT)
        def _(j):
            pltpu.sync_copy(ones_v, acc_sh.at[idx_v.at[j]], add=True)

        plsc.subcore_barrier()
        pltpu.sync_copy(acc_sh.at[pl.ds(t * ZR, ZR)],
                        out_hbm.at[c, pl.ds(t * ZR, ZR)])

    return deg_kernel(srcdst_rows)


def _sc_aggregate(table, src_rows, dst_rows):
    @functools.partial(
        pl.kernel,
        out_type=jax.ShapeDtypeStruct((NC, NPAD, 128), jnp.float32),
        mesh=plsc.VectorSubcoreMesh(**_MESH),
        scratch_types=[
            pltpu.VMEM((RPW, 128), jnp.int32),
            pltpu.VMEM((RPW, 128), jnp.int32),
            pltpu.VMEM((128, 128), jnp.float32),
            pltpu.VMEM_SHARED((NPAD, 128), jnp.float32),
        ],
    )
    def agg_kernel(table_hbm, src_hbm, dst_hbm, out_hbm, idx_s, idx_d, rows_v,
                   acc_sh):
        c = lax.axis_index("c")
        t = lax.axis_index("s")
        wid = c * NS + t

        @pl.loop(0, 128)
        def _(r):
            @pl.loop(0, 8)
            def _(q):
                rows_v[r, pl.ds(q * 16, 16)] = jnp.zeros((16,), jnp.float32)

        for q in range(ZR // 128):
            pltpu.sync_copy(rows_v,
                            acc_sh.at[pl.ds(t * ZR + q * 128, 128)])
        plsc.subcore_barrier()

        base = wid * RPW
        pltpu.sync_copy(src_hbm.at[pl.ds(base, RPW)], idx_s)
        pltpu.sync_copy(dst_hbm.at[pl.ds(base, RPW)], idx_d)

        @pl.loop(0, RPW)
        def _(j):
            pltpu.sync_copy(table_hbm.at[idx_s.at[j]], rows_v)
            pltpu.sync_copy(rows_v, acc_sh.at[idx_d.at[j]], add=True)

        plsc.subcore_barrier()
        pltpu.sync_copy(acc_sh.at[pl.ds(t * ZR, ZR)],
                        out_hbm.at[c, pl.ds(t * ZR, ZR)])

    return agg_kernel(table, src_rows, dst_rows)


_BR = 1000
_DOT = dict(precision=lax.Precision.HIGHEST,
            preferred_element_type=jnp.float32)


def _tc_front(f1, f2, f3, f4, f5, W1, W2, W3, W4, W5, b1, b2, b3, b4, b5,
              w1, w2, w3, w4, w5, Wg1):
    def body(f1r, f2r, f3r, f4r, f5r, W1r, W2r, W3r, W4r, W5r,
             b1r, b2r, b3r, b4r, b5r, wr, Wg1r, out):
        w = wr[0]
        h = jnp.dot(f1r[...], W1r[...], **_DOT) * w[0]
        h += jnp.dot(f2r[...], W2r[...], **_DOT) * w[1]
        h += jnp.dot(f3r[...], W3r[...], **_DOT) * w[2]
        h += jnp.dot(f4r[...], W4r[...], **_DOT) * w[3]
        h += jnp.dot(f5r[...], W5r[...], **_DOT) * w[4]
        h += (b1r[...] * w[0] + b2r[...] * w[1] + b3r[...] * w[2]
              + b4r[...] * w[3] + b5r[...] * w[4])
        out[...] = jnp.dot(h, Wg1r[...], **_DOT)

    full = lambda a: pl.BlockSpec(a.shape, lambda i: (0,) * a.ndim)
    row = lambda a: pl.BlockSpec((_BR, a.shape[1]), lambda i: (i, 0))
    wvec = jnp.stack([w1[0], w2[0], w3[0], w4[0], w5[0]]).reshape(1, 5)
    bs = [b.reshape(1, 128) for b in (b1, b2, b3, b4, b5)]
    return pl.pallas_call(
        body,
        grid=(N // _BR,),
        in_specs=[row(f1), row(f2), row(f3), row(f4), row(f5),
                  full(W1), full(W2), full(W3), full(W4), full(W5),
                  full(bs[0]), full(bs[1]), full(bs[2]), full(bs[3]),
                  full(bs[4]), full(wvec), full(Wg1)],
        out_specs=pl.BlockSpec((_BR, 128), lambda i: (i, 0)),
        out_shape=jax.ShapeDtypeStruct((N, 128), jnp.float32),
    )(f1, f2, f3, f4, f5, W1, W2, W3, W4, W5, *bs, wvec, Wg1)


def _rs(deg_block):
    return lax.rsqrt(jnp.maximum(deg_block, 1.0))


def _tc_scale(z, od):
    def body(zr, dr, out):
        out[...] = zr[...] * _rs(dr[...])

    return pl.pallas_call(
        body,
        grid=(N // _BR,),
        in_specs=[pl.BlockSpec((_BR, 128), lambda i: (i, 0)),
                  pl.BlockSpec((_BR, 1), lambda i: (i, 0))],
        out_specs=pl.BlockSpec((_BR, 128), lambda i: (i, 0)),
        out_shape=jax.ShapeDtypeStruct((N, 128), jnp.float32),
    )(z, od)


def _tc_mid(agg, id_, od, bg1, Wg2):
    def body(a0, a1, dr_in, dr_out, br, Wr, out):
        s = (a0[0] + a1[0]) * _rs(dr_in[...]) + br[...]
        h2 = jnp.maximum(s, 0.0)
        out[...] = jnp.dot(h2, Wr[...], **_DOT) * _rs(dr_out[...])

    return pl.pallas_call(
        body,
        grid=(N // _BR,),
        in_specs=[pl.BlockSpec((1, _BR, 128), lambda i: (0, i, 0)),
                  pl.BlockSpec((1, _BR, 128), lambda i: (1, i, 0)),
                  pl.BlockSpec((_BR, 1), lambda i: (i, 0)),
                  pl.BlockSpec((_BR, 1), lambda i: (i, 0)),
                  pl.BlockSpec((1, 128), lambda i: (0, 0)),
                  pl.BlockSpec((128, 128), lambda i: (0, 0))],
        out_specs=pl.BlockSpec((_BR, 128), lambda i: (i, 0)),
        out_shape=jax.ShapeDtypeStruct((N, 128), jnp.float32),
    )(agg, agg, id_, od, bg1.reshape(1, 128), Wg2)


def _tc_tail(agg, id_, bg2, Wb1, bb1, Wb2, bb2):
    def body(a0, a1, dr_in, br, W1r, b1r, W2r, b2r, out):
        s = (a0[0] + a1[0]) * _rs(dr_in[...]) + br[...]
        tm = jnp.dot(s, W1r[...], **_DOT) + b1r[...]
        out[...] = (jnp.sum(tm * W2r[...], axis=1) + b2r[0, 0])[:, None]

    return pl.pallas_call(
        body,
        grid=(N // _BR,),
        in_specs=[pl.BlockSpec((1, _BR, 128), lambda i: (0, i, 0)),
                  pl.BlockSpec((1, _BR, 128), lambda i: (1, i, 0)),
                  pl.BlockSpec((_BR, 1), lambda i: (i, 0)),
                  pl.BlockSpec((1, 128), lambda i: (0, 0)),
                  pl.BlockSpec((128, 64), lambda i: (0, 0)),
                  pl.BlockSpec((1, 64), lambda i: (0, 0)),
                  pl.BlockSpec((1, 64), lambda i: (0, 0)),
                  pl.BlockSpec((1, 1), lambda i: (0, 0))],
        out_specs=pl.BlockSpec((_BR, 1), lambda i: (i, 0)),
        out_shape=jax.ShapeDtypeStruct((N, 1), jnp.float32),
    )(agg, agg, id_, bg2.reshape(1, 128), Wb1, bb1.reshape(1, 64),
      Wb2.reshape(1, 64), bb2.reshape(1, 1))


def kernel(features1, features2, features3, features4, features5, edge_index,
           Wf1, bf1, Wf2, bf2, Wf3, bf3, Wf4, bf4, Wf5, bf5,
           w1, w2, w3, w4, w5, Wg1, bg1, Wg2, bg2, Wb1, bb1, Wb2, bb2):
    src, dst = edge_index[0], edge_index[1]
    trash = jnp.full((EPAD,), N, jnp.int32)
    src_agg = jnp.concatenate([src, jnp.zeros((EPAD,), jnp.int32)]
                              ).reshape(EROWS, 128)
    dst_rows = jnp.concatenate([dst, trash]).reshape(EROWS, 128)
    srcdst = jnp.concatenate(
        [jnp.concatenate([src, trash]).reshape(EROWS, 128), dst_rows])

    degs = _sc_degrees_spmem(srcdst)
    od, id_ = degs[0][:, None], degs[1][:, None]

    z1pre = _tc_front(features1, features2, features3, features4, features5,
                      Wf1, Wf2, Wf3, Wf4, Wf5, bf1, bf2, bf3, bf4, bf5,
                      w1, w2, w3, w4, w5, Wg1)
    z1 = _tc_scale(z1pre, od)
    agg1 = _sc_aggregate(z1, src_agg, dst_rows)
    z2 = _tc_mid(agg1, id_, od, bg1, Wg2)
    agg2 = _sc_aggregate(z2, src_agg, dst_rows)
    return _tc_tail(agg2, id_, bg2, Wb1, bb1, Wb2, bb2).reshape(-1)


# speedup vs baseline: 3.8595x; 1.0810x over previous
import functools

import jax
import jax.numpy as jnp
from jax import lax
from jax.experimental import pallas as pl
from jax.experimental.pallas import tpu as pltpu
from jax.experimental.pallas import tpu_sc as plsc

N = 10000
E = 320000
NPAD = 10240
NAGG = 10112
EROWS = 2560
EPAD = EROWS * 128 - E
NC, NS = 2, 16
RPW = EROWS // (NC * NS)
RPT = EROWS // NS
ZR = NPAD // NS
ZRA = NAGG // NS

_MESH = dict(core_axis_name="c", subcore_axis_name="s", num_cores=NC,
             num_subcores=NS)


def _sc_degrees_spmem(srcdst_rows):
    @functools.partial(
        pl.kernel,
        out_type=jax.ShapeDtypeStruct((2, NPAD), jnp.float32),
        mesh=plsc.VectorSubcoreMesh(**_MESH),
        scratch_types=[
            pltpu.VMEM((RPT, 128), jnp.int32),
            pltpu.VMEM((128,), jnp.float32),
            pltpu.VMEM((ZR,), jnp.float32),
            pltpu.VMEM_SHARED((NPAD,), jnp.float32),
        ],
    )
    def deg_kernel(se_hbm, out_hbm, idx_v, ones_v, z_v, acc_sh):
        c = lax.axis_index("c")
        t = lax.axis_index("s")

        @pl.loop(0, 8)
        def _(q):
            ones_v[pl.ds(q * 16, 16)] = jnp.full((16,), 1.0, jnp.float32)

        @pl.loop(0, ZR // 16)
        def _(q):
            z_v[pl.ds(q * 16, 16)] = jnp.zeros((16,), jnp.float32)

        pltpu.sync_copy(z_v, acc_sh.at[pl.ds(t * ZR, ZR)])
        plsc.subcore_barrier()

        pltpu.sync_copy(se_hbm.at[pl.ds(c * EROWS + t * RPT, RPT)], idx_v)

        @pl.loop(0, RPT)
        def _(j):
            pltpu.sync_copy(ones_v, acc_sh.at[idx_v.at[j]], add=True)

        plsc.subcore_barrier()
        pltpu.sync_copy(acc_sh.at[pl.ds(t * ZR, ZR)],
                        out_hbm.at[c, pl.ds(t * ZR, ZR)])

    return deg_kernel(srcdst_rows)


def _sc_aggregate(table, src_rows, dst_rows):
    HALF = RPW // 2

    @functools.partial(
        pl.kernel,
        out_type=jax.ShapeDtypeStruct((NC, NAGG, 128), jnp.float32),
        mesh=plsc.VectorSubcoreMesh(**_MESH),
        scratch_types=[
            pltpu.VMEM((HALF + 8, 128), jnp.int32),
            pltpu.VMEM((HALF, 128), jnp.int32),
            pltpu.VMEM((128, 128), jnp.float32),
            pltpu.VMEM((128, 128), jnp.float32),
            pltpu.SemaphoreType.DMA,
            pltpu.SemaphoreType.DMA,
            pltpu.VMEM_SHARED((NAGG, 128), jnp.float32),
        ],
    )
    def agg_kernel(table_hbm, src_hbm, dst_hbm, out_hbm, idx_s, idx_d,
                   rows0, rows1, sem0, sem1, acc_sh):
        c = lax.axis_index("c")
        t = lax.axis_index("s")
        wid = c * NS + t

        @pl.loop(0, 128)
        def _(r):
            @pl.loop(0, 8)
            def _(q):
                rows0[r, pl.ds(q * 16, 16)] = jnp.zeros((16,), jnp.float32)

        for q in range(ZRA // 128):
            pltpu.sync_copy(rows0,
                            acc_sh.at[pl.ds(t * ZRA + q * 128, 128)])
        pltpu.sync_copy(rows0.at[pl.ds(0, ZRA % 128)],
                        acc_sh.at[pl.ds(t * ZRA + ZRA - ZRA % 128,
                                        ZRA % 128)])
        plsc.subcore_barrier()

        base = wid * RPW

        def run_half(off):
            pltpu.sync_copy(src_hbm.at[pl.ds(base + off, HALF)],
                            idx_s.at[pl.ds(0, HALF)])
            pltpu.sync_copy(src_hbm.at[pl.ds(base, 8)],
                            idx_s.at[pl.ds(HALF, 8)])
            pltpu.sync_copy(dst_hbm.at[pl.ds(base + off, HALF)], idx_d)

            pltpu.async_copy(table_hbm.at[idx_s.at[0]], rows0, sem0)

            @pl.loop(0, HALF // 2)
            def _(i):
                j0 = 2 * i
                pltpu.make_async_copy(table_hbm.at[idx_s.at[j0]], rows0,
                                      sem0).wait()
                pltpu.async_copy(table_hbm.at[idx_s.at[j0 + 1]], rows1, sem1)
                pltpu.sync_copy(rows0, acc_sh.at[idx_d.at[j0]], add=True)
                pltpu.make_async_copy(table_hbm.at[idx_s.at[j0 + 1]], rows1,
                                      sem1).wait()
                pltpu.async_copy(table_hbm.at[idx_s.at[j0 + 2]], rows0, sem0)
                pltpu.sync_copy(rows1, acc_sh.at[idx_d.at[j0 + 1]], add=True)

            pltpu.make_async_copy(table_hbm.at[idx_s.at[HALF]], rows0,
                                  sem0).wait()

        run_half(0)
        run_half(HALF)

        plsc.subcore_barrier()
        pltpu.sync_copy(acc_sh.at[pl.ds(t * ZRA, ZRA)],
                        out_hbm.at[c, pl.ds(t * ZRA, ZRA)])

    return agg_kernel(table, src_rows, dst_rows)


_BR = 1000
_DOT = dict(precision=lax.Precision.HIGHEST,
            preferred_element_type=jnp.float32)


def _tc_front(f1, f2, f3, f4, f5, W1, W2, W3, W4, W5, b1, b2, b3, b4, b5,
              w1, w2, w3, w4, w5, Wg1):
    def body(f1r, f2r, f3r, f4r, f5r, W1r, W2r, W3r, W4r, W5r,
             b1r, b2r, b3r, b4r, b5r, wr, Wg1r, out):
        w = wr[0]
        h = jnp.dot(f1r[...], W1r[...], **_DOT) * w[0]
        h += jnp.dot(f2r[...], W2r[...], **_DOT) * w[1]
        h += jnp.dot(f3r[...], W3r[...], **_DOT) * w[2]
        h += jnp.dot(f4r[...], W4r[...], **_DOT) * w[3]
        h += jnp.dot(f5r[...], W5r[...], **_DOT) * w[4]
        h += (b1r[...] * w[0] + b2r[...] * w[1] + b3r[...] * w[2]
              + b4r[...] * w[3] + b5r[...] * w[4])
        out[...] = jnp.dot(h, Wg1r[...], **_DOT)

    full = lambda a: pl.BlockSpec(a.shape, lambda i: (0,) * a.ndim)
    row = lambda a: pl.BlockSpec((_BR, a.shape[1]), lambda i: (i, 0))
    wvec = jnp.stack([w1[0], w2[0], w3[0], w4[0], w5[0]]).reshape(1, 5)
    bs = [b.reshape(1, 128) for b in (b1, b2, b3, b4, b5)]
    return pl.pallas_call(
        body,
        grid=(N // _BR,),
        in_specs=[row(f1), row(f2), row(f3), row(f4), row(f5),
                  full(W1), full(W2), full(W3), full(W4), full(W5),
                  full(bs[0]), full(bs[1]), full(bs[2]), full(bs[3]),
                  full(bs[4]), full(wvec), full(Wg1)],
        out_specs=pl.BlockSpec((_BR, 128), lambda i: (i, 0)),
        out_shape=jax.ShapeDtypeStruct((N, 128), jnp.float32),
    )(f1, f2, f3, f4, f5, W1, W2, W3, W4, W5, *bs, wvec, Wg1)


def _rs(deg_block):
    return lax.rsqrt(jnp.maximum(deg_block, 1.0))


def _tc_scale(z, od):
    def body(zr, dr, out):
        out[...] = zr[...] * _rs(dr[...])

    return pl.pallas_call(
        body,
        grid=(N // _BR,),
        in_specs=[pl.BlockSpec((_BR, 128), lambda i: (i, 0)),
                  pl.BlockSpec((_BR, 1), lambda i: (i, 0))],
        out_specs=pl.BlockSpec((_BR, 128), lambda i: (i, 0)),
        out_shape=jax.ShapeDtypeStruct((N, 128), jnp.float32),
    )(z, od)


def _tc_mid(agg, id_, od, bg1, Wg2):
    def body(a0, a1, dr_in, dr_out, br, Wr, out):
        s = (a0[0] + a1[0]) * _rs(dr_in[...]) + br[...]
        h2 = jnp.maximum(s, 0.0)
        out[...] = jnp.dot(h2, Wr[...], **_DOT) * _rs(dr_out[...])

    return pl.pallas_call(
        body,
        grid=(N // _BR,),
        in_specs=[pl.BlockSpec((1, _BR, 128), lambda i: (0, i, 0)),
                  pl.BlockSpec((1, _BR, 128), lambda i: (1, i, 0)),
                  pl.BlockSpec((_BR, 1), lambda i: (i, 0)),
                  pl.BlockSpec((_BR, 1), lambda i: (i, 0)),
                  pl.BlockSpec((1, 128), lambda i: (0, 0)),
                  pl.BlockSpec((128, 128), lambda i: (0, 0))],
        out_specs=pl.BlockSpec((_BR, 128), lambda i: (i, 0)),
        out_shape=jax.ShapeDtypeStruct((N, 128), jnp.float32),
    )(agg, agg, id_, od, bg1.reshape(1, 128), Wg2)


def _tc_tail(agg, id_, bg2, Wb1, bb1, Wb2, bb2):
    def body(a0, a1, dr_in, br, W1r, b1r, W2r, b2r, out):
        s = (a0[0] + a1[0]) * _rs(dr_in[...]) + br[...]
        tm = jnp.dot(s, W1r[...], **_DOT) + b1r[...]
        out[...] = (jnp.sum(tm * W2r[...], axis=1) + b2r[0, 0])[:, None]

    return pl.pallas_call(
        body,
        grid=(N // _BR,),
        in_specs=[pl.BlockSpec((1, _BR, 128), lambda i: (0, i, 0)),
                  pl.BlockSpec((1, _BR, 128), lambda i: (1, i, 0)),
                  pl.BlockSpec((_BR, 1), lambda i: (i, 0)),
                  pl.BlockSpec((1, 128), lambda i: (0, 0)),
                  pl.BlockSpec((128, 64), lambda i: (0, 0)),
                  pl.BlockSpec((1, 64), lambda i: (0, 0)),
                  pl.BlockSpec((1, 64), lambda i: (0, 0)),
                  pl.BlockSpec((1, 1), lambda i: (0, 0))],
        out_specs=pl.BlockSpec((_BR, 1), lambda i: (i, 0)),
        out_shape=jax.ShapeDtypeStruct((N, 1), jnp.float32),
    )(agg, agg, id_, bg2.reshape(1, 128), Wb1, bb1.reshape(1, 64),
      Wb2.reshape(1, 64), bb2.reshape(1, 1))


def kernel(features1, features2, features3, features4, features5, edge_index,
           Wf1, bf1, Wf2, bf2, Wf3, bf3, Wf4, bf4, Wf5, bf5,
           w1, w2, w3, w4, w5, Wg1, bg1, Wg2, bg2, Wb1, bb1, Wb2, bb2):
    src, dst = edge_index[0], edge_index[1]
    trash = jnp.full((EPAD,), N, jnp.int32)
    src_agg = jnp.concatenate([src, jnp.zeros((EPAD,), jnp.int32)]
                              ).reshape(EROWS, 128)
    dst_rows = jnp.concatenate([dst, trash]).reshape(EROWS, 128)
    srcdst = jnp.concatenate(
        [jnp.concatenate([src, trash]).reshape(EROWS, 128), dst_rows])

    degs = _sc_degrees_spmem(srcdst)
    od, id_ = degs[0][:, None], degs[1][:, None]

    z1pre = _tc_front(features1, features2, features3, features4, features5,
                      Wf1, Wf2, Wf3, Wf4, Wf5, bf1, bf2, bf3, bf4, bf5,
                      w1, w2, w3, w4, w5, Wg1)
    z1 = _tc_scale(z1pre, od)
    agg1 = _sc_aggregate(z1, src_agg, dst_rows)
    z2 = _tc_mid(agg1, id_, od, bg1, Wg2)
    agg2 = _sc_aggregate(z2, src_agg, dst_rows)
    return _tc_tail(agg2, id_, bg2, Wb1, bb1, Wb2, bb2).reshape(-1)
